# baseline (speedup 1.0000x reference)
import functools

import jax
import jax.numpy as jnp
from jax import lax
from jax.experimental import pallas as pl
from jax.experimental.pallas import tpu as pltpu
from jax.experimental.pallas import tpu_sc as plsc

D_MODEL = 256
T_ROWS = 256
NUM_CORES = 2
NUM_SUBCORES = 16
NUM_WORKERS = NUM_CORES * NUM_SUBCORES
NBUF = 2
KB = 4
SEQ_STRIDE = 64
GP = 56


def _prep_kernel(card_ref, stg_ref, rank_ref, suit_ref, stage_ref,
                 t_ref, idx_ref):
    rows = lax.broadcasted_iota(jnp.int32, (T_ROWS, 1), 0)
    card = rows // 4
    stg = rows % 4
    rank = card % 13
    suit = card // 13
    valid = card < 52
    acc = jnp.zeros((T_ROWS, D_MODEL), jnp.float32)
    for k in range(13):
        acc += jnp.where(rank == k, 1.0, 0.0) * rank_ref[k, :][None, :]
    for k in range(4):
        acc += jnp.where(suit == k, 1.0, 0.0) * suit_ref[k, :][None, :]
        acc += jnp.where(stg == k, 1.0, 0.0) * stage_ref[k, :][None, :]
    t_ref[...] = jnp.where(valid, acc, 0.0)

    c = card_ref[...]
    s = stg_ref[...]
    cvalid = (c >= 0) & (c < 52)
    cc = jnp.where(cvalid, c, 52)
    ss = jnp.clip(s, 0, 3)
    idx_ref[...] = cc * 4 + ss


def _prep(card_indices, stages, rank_emb, suit_emb, stage_emb):
    batch, seq = card_indices.shape
    return pl.pallas_call(
        _prep_kernel,
        out_shape=(
            jax.ShapeDtypeStruct((T_ROWS, D_MODEL), jnp.float32),
            jax.ShapeDtypeStruct((batch, seq), jnp.int32),
        ),
    )(card_indices, stages, rank_emb, suit_emb, stage_emb)


def _make_sc_gather(batch, seq):
    n_chunks = batch // KB
    assert batch % (NUM_WORKERS * NBUF * KB) == 0
    chunks_per_worker = n_chunks // NUM_WORKERS
    cw = KB * SEQ_STRIDE
    mesh = plsc.VectorSubcoreMesh(core_axis_name="c", subcore_axis_name="s")

    scratch = []
    for _ in range(NBUF):
        scratch += [
            pltpu.VMEM((cw,), jnp.int32),
            pltpu.VMEM((KB, GP, D_MODEL), jnp.float32),
            pltpu.SemaphoreType.DMA,
            pltpu.SemaphoreType.DMA,
            pltpu.SemaphoreType.DMA,
        ]

    @functools.partial(
        pl.kernel,
        out_type=jax.ShapeDtypeStruct((batch, GP, D_MODEL), jnp.float32),
        mesh=mesh,
        scratch_types=scratch,
    )
    def sc_gather(idx_hbm, t_hbm, out_hbm, *bufs):
        idx_v = [bufs[5 * b + 0] for b in range(NBUF)]
        rows_v = [bufs[5 * b + 1] for b in range(NBUF)]
        isem = [bufs[5 * b + 2] for b in range(NBUF)]
        gsem = [bufs[5 * b + 3] for b in range(NBUF)]
        osem = [bufs[5 * b + 4] for b in range(NBUF)]
        wid = lax.axis_index("s") * NUM_CORES + lax.axis_index("c")
        chunk0 = wid * chunks_per_worker

        for b in range(NBUF):
            pltpu.async_copy(idx_hbm.at[chunk0 + b], idx_v[b], isem[b])

        def group(g, carry):
            for b in range(NBUF):
                r = g * NBUF + b
                pltpu.make_async_copy(
                    idx_hbm.at[chunk0], idx_v[b], isem[b]).wait()

                @pl.when(r >= NBUF)
                def _rows_free():
                    pltpu.make_async_copy(
                        rows_v[b], out_hbm.at[pl.ds(0, KB)], osem[b]).wait()

                for j in range(KB):
                    pltpu.async_copy(
                        t_hbm.at[idx_v[b].at[pl.ds(j * SEQ_STRIDE, GP)]],
                        rows_v[b].at[j], gsem[b])
                for j in range(KB):
                    pltpu.make_async_copy(
                        t_hbm.at[idx_v[b].at[pl.ds(0, GP)]],
                        rows_v[b].at[0], gsem[b]).wait()

                @pl.when(r + NBUF < chunks_per_worker)
                def _next_idx():
                    pltpu.async_copy(
                        idx_hbm.at[chunk0 + r + NBUF], idx_v[b], isem[b])

                pltpu.async_copy(
                    rows_v[b],
                    out_hbm.at[pl.ds((chunk0 + r) * KB, KB)], osem[b])
            return carry

        lax.fori_loop(0, chunks_per_worker // NBUF, group, 0)
        for b in range(NBUF):
            pltpu.make_async_copy(
                rows_v[b], out_hbm.at[pl.ds(0, KB)], osem[b]).wait()

    return sc_gather


def kernel(card_indices, stages, rank_emb, suit_emb, stage_emb):
    batch, seq = card_indices.shape
    table, idx = _prep(card_indices.astype(jnp.int32),
                       stages.astype(jnp.int32),
                       rank_emb, suit_emb, stage_emb)
    idx_p = jnp.pad(idx, ((0, 0), (0, SEQ_STRIDE - seq)))
    idx_c = idx_p.reshape(batch // KB, KB * SEQ_STRIDE)
    out = _make_sc_gather(batch, seq)(idx_c, table)
    return out[:, :seq, :]
